# baseline (speedup 1.0000x reference)
import functools

import jax
import jax.numpy as jnp
from jax import lax
from jax.experimental import pallas as pl
from jax.experimental.pallas import tpu as pltpu
from jax.experimental.pallas import tpu_sc as plsc

VOCAB = 1000000
EMBED_DIM = 64
TOTAL = 16384 * 20
NUM_WORKERS = 32
BPW = TOTAL // NUM_WORKERS
CHUNK = 512
NCHUNK = BPW // CHUNK

SLAB_W = 256
N_SLABS = 3906
NI = 123


def _transpose_table(weight_t, weight_tail):
    mesh = plsc.VectorSubcoreMesh(core_axis_name="c", subcore_axis_name="s")

    @functools.partial(
        pl.kernel,
        mesh=mesh,
        out_type=jax.ShapeDtypeStruct((VOCAB // 2, 128), jnp.float32),
        scratch_types=[
            pltpu.VMEM((64, SLAB_W), jnp.float32),
            pltpu.VMEM((64, SLAB_W), jnp.float32),
            pltpu.VMEM((SLAB_W // 2, 128), jnp.float32),
            pltpu.VMEM((SLAB_W // 2, 128), jnp.float32),
            pltpu.VMEM((64, 128), jnp.float32),
            pltpu.VMEM((32, 128), jnp.float32),
            pltpu.SemaphoreType.DMA,
            pltpu.SemaphoreType.DMA,
            pltpu.SemaphoreType.DMA,
            pltpu.SemaphoreType.DMA,
        ],
        compiler_params=pltpu.CompilerParams(
            use_tc_tiling_on_sc=True, needs_layout_passes=False),
    )
    def k(wt_hbm, wtail_hbm, out_hbm, in0_v, in1_v, tr0_v, tr1_v,
          tin_v, ttr_v, r0, r1, w0, w1):
        wid = lax.axis_index("s") * 2 + lax.axis_index("c")
        iota16 = lax.iota(jnp.int32, 16)
        in_b = (in0_v, in1_v)
        tr_b = (tr0_v, tr1_v)
        rsem = (r0, r1)
        wsem = (w0, w1)

        def s_of(i):
            return wid + NUM_WORKERS * i

        def valid(i):
            return s_of(i) < N_SLABS

        def read_issue(i, b):
            for tr in range(8):
                pltpu.async_copy(
                    wt_hbm.at[pl.ds(8 * tr, 8),
                              pl.ds(s_of(i) * SLAB_W, SLAB_W)],
                    in_b[b].at[pl.ds(8 * tr, 8), :], rsem[b])

        def read_wait(b):
            pltpu.make_async_copy(
                wt_hbm.at[:, pl.ds(0, SLAB_W)], in_b[b], rsem[b]).wait()

        def write_issue(i, b):
            pltpu.async_copy(
                tr_b[b],
                out_hbm.at[pl.ds(s_of(i) * (SLAB_W // 2), SLAB_W // 2), :],
                wsem[b])

        def write_wait(b):
            pltpu.make_async_copy(
                tr_b[b],
                out_hbm.at[pl.ds(0, SLAB_W // 2), :], wsem[b]).wait()

        def transpose(b):
            def v_body(vb, c):
                v0 = vb * 16
                u0 = vb * 8
                for fj in range(4):
                    for d in range(16):
                        r = (iota16 + d) & 15
                        fv = iota16 + (16 * fj)
                        vals = plsc.load_gather(in_b[b], [fv, r + v0])
                        plsc.store_scatter(
                            tr_b[b],
                            [lax.shift_right_logical(r, 1) + u0,
                             lax.shift_left(r & 1, 6) + fv],
                            vals)
                return c

            lax.fori_loop(0, SLAB_W // 16, v_body, 0)

        @pl.when(valid(0))
        def _prime():
            read_issue(0, 0)

        def pair_body(t, carry):
            for b in range(2):
                i = 2 * t + b

                @pl.when(valid(i))
                def _(i=i, b=b):
                    read_wait(b)

                    @pl.when(valid(i + 1))
                    def _():
                        read_issue(i + 1, 1 - b)

                    @pl.when(i >= 2)
                    def _():
                        write_wait(b)

                    transpose(b)
                    write_issue(i, b)

            return carry

        lax.fori_loop(0, (NI + 1) // 2, pair_body, 0)
        write_wait(0)
        write_wait(1)

        @pl.when(wid == NUM_WORKERS - 1)
        def _tail():
            pltpu.sync_copy(wtail_hbm, tin_v)

            def u_body(u, c):
                for j in range(8):
                    f_idx = ((16 * j) % 64) + iota16
                    col = iota16 * 0 + (2 * u + (j // 4))
                    ttr_v[u, pl.ds(16 * j, 16)] = plsc.load_gather(
                        tin_v, [f_idx, col])
                return c

            lax.fori_loop(0, 32, u_body, 0)
            pltpu.sync_copy(
                ttr_v, out_hbm.at[pl.ds(N_SLABS * (SLAB_W // 2), 32), :])

    return k(weight_t, weight_tail)


def _emb_lookup(idx_flat, table_lin):
    mesh = plsc.VectorSubcoreMesh(core_axis_name="c", subcore_axis_name="s")

    @functools.partial(
        pl.kernel,
        mesh=mesh,
        out_type=jax.ShapeDtypeStruct((TOTAL, EMBED_DIM), jnp.float32),
        scratch_types=[
            pltpu.VMEM((BPW,), jnp.int32),
            pltpu.VMEM((2, CHUNK, EMBED_DIM), jnp.float32),
            pltpu.SemaphoreType.DMA,
            pltpu.SemaphoreType.DMA,
        ],
        compiler_params=pltpu.CompilerParams(use_tc_tiling_on_sc=False),
    )
    def k(idx_hbm, table_hbm, out_hbm, idx_v, rows_v, gsem, wsem):
        wid = lax.axis_index("s") * 2 + lax.axis_index("c")
        base = wid * BPW
        pltpu.sync_copy(idx_hbm.at[pl.ds(base, BPW)], idx_v)

        def gather(c, b):
            return pltpu.async_copy(
                table_hbm.at[idx_v.at[pl.ds(c * CHUNK, CHUNK)]],
                rows_v.at[b], gsem)

        def write(c, b):
            return pltpu.async_copy(
                rows_v.at[b], out_hbm.at[pl.ds(base + c * CHUNK, CHUNK)], wsem)

        writes = [None] * NCHUNK
        g = gather(0, 0)
        for c in range(NCHUNK):
            b = c % 2
            if c + 1 < NCHUNK:
                if c >= 1:
                    writes[c - 1].wait()
                g_next = gather(c + 1, (c + 1) % 2)
            g.wait()
            writes[c] = write(c, b)
            if c + 1 < NCHUNK:
                g = g_next
        writes[NCHUNK - 2].wait()
        writes[NCHUNK - 1].wait()

    return k(idx_flat, table_lin)


def kernel(input, weight):
    idx = input.reshape(-1).astype(jnp.int32)
    wtail = jnp.pad(weight[VOCAB - 64:].T, ((0, 0), (0, 64)))
    table_fused = _transpose_table(weight.T, wtail)
    table_lin = table_fused.reshape(VOCAB, EMBED_DIM)
    out = _emb_lookup(idx, table_lin)
    return out.reshape(input.shape + (EMBED_DIM,))


# speedup vs baseline: 1.0186x; 1.0186x over previous
import functools

import jax
import jax.numpy as jnp
from jax import lax
from jax.experimental import pallas as pl
from jax.experimental.pallas import tpu as pltpu
from jax.experimental.pallas import tpu_sc as plsc

VOCAB = 1000000
EMBED_DIM = 64
TOTAL = 16384 * 20
NUM_WORKERS = 32
BPW = TOTAL // NUM_WORKERS
CHUNK = 512
NCHUNK = BPW // CHUNK

SLAB_W = 256
N_SLABS = 3906
NI = 123


def _transpose_table(weight_t, weight_tail):
    mesh = plsc.VectorSubcoreMesh(core_axis_name="c", subcore_axis_name="s")

    @functools.partial(
        pl.kernel,
        mesh=mesh,
        out_type=jax.ShapeDtypeStruct((VOCAB // 2, 128), jnp.float32),
        scratch_types=[
            pltpu.VMEM((64, SLAB_W), jnp.float32),
            pltpu.VMEM((64, SLAB_W), jnp.float32),
            pltpu.VMEM((SLAB_W // 2, 128), jnp.float32),
            pltpu.VMEM((SLAB_W // 2, 128), jnp.float32),
            pltpu.VMEM((64, 128), jnp.float32),
            pltpu.VMEM((32, 128), jnp.float32),
            pltpu.SemaphoreType.DMA,
            pltpu.SemaphoreType.DMA,
            pltpu.SemaphoreType.DMA,
            pltpu.SemaphoreType.DMA,
        ],
        compiler_params=pltpu.CompilerParams(
            use_tc_tiling_on_sc=True, needs_layout_passes=False),
    )
    def k(wt_hbm, wtail_hbm, out_hbm, in0_v, in1_v, tr0_v, tr1_v,
          tin_v, ttr_v, r0, r1, w0, w1):
        wid = lax.axis_index("s") * 2 + lax.axis_index("c")
        iota16 = lax.iota(jnp.int32, 16)
        in_b = (in0_v, in1_v)
        tr_b = (tr0_v, tr1_v)
        rsem = (r0, r1)
        wsem = (w0, w1)

        def s_of(i):
            return wid + NUM_WORKERS * i

        def valid(i):
            return s_of(i) < N_SLABS

        def read_issue(i, b):
            for tr in range(8):
                pltpu.async_copy(
                    wt_hbm.at[pl.ds(8 * tr, 8),
                              pl.ds(s_of(i) * SLAB_W, SLAB_W)],
                    in_b[b].at[pl.ds(8 * tr, 8), :], rsem[b])

        def read_wait(b):
            pltpu.make_async_copy(
                wt_hbm.at[:, pl.ds(0, SLAB_W)], in_b[b], rsem[b]).wait()

        def write_issue(i, b):
            pltpu.async_copy(
                tr_b[b],
                out_hbm.at[pl.ds(s_of(i) * (SLAB_W // 2), SLAB_W // 2), :],
                wsem[b])

        def write_wait(b):
            pltpu.make_async_copy(
                tr_b[b],
                out_hbm.at[pl.ds(0, SLAB_W // 2), :], wsem[b]).wait()

        lane = iota16
        pvec = lax.shift_right_logical(lane, 3)
        p64v = lax.shift_left(pvec, 6)
        foct8 = lax.shift_left(lane & 7, 3)
        fvecs = [foct8 + f_in for f_in in range(8)]
        coutv = [p64v + fv for fv in fvecs]

        def transpose(b):
            def t_body(t, c):
                ub64 = lax.shift_left(lax.shift_right_logical(t, 4), 6)
                d = t & 15
                g4 = lax.shift_left((lane + d) & 15, 2)
                for u0 in range(4):
                    uvec = g4 + (ub64 + u0)
                    vvec = lax.shift_left(uvec, 1) + pvec
                    for f_in in range(8):
                        vals = plsc.load_gather(
                            in_b[b], [fvecs[f_in], vvec])
                        plsc.store_scatter(
                            tr_b[b], [uvec, coutv[f_in]], vals)
                return c

            lax.fori_loop(0, (SLAB_W // 128) * 16, t_body, 0)

        @pl.when(valid(0))
        def _prime():
            read_issue(0, 0)

        def pair_body(t, carry):
            for b in range(2):
                i = 2 * t + b

                @pl.when(valid(i))
                def _(i=i, b=b):
                    read_wait(b)

                    @pl.when(valid(i + 1))
                    def _():
                        read_issue(i + 1, 1 - b)

                    @pl.when(i >= 2)
                    def _():
                        write_wait(b)

                    transpose(b)
                    write_issue(i, b)

            return carry

        lax.fori_loop(0, (NI + 1) // 2, pair_body, 0)
        write_wait(0)
        write_wait(1)

        @pl.when(wid == NUM_WORKERS - 1)
        def _tail():
            pltpu.sync_copy(wtail_hbm, tin_v)

            def u_body(u, c):
                for j in range(8):
                    f_idx = ((16 * j) % 64) + iota16
                    col = iota16 * 0 + (2 * u + (j // 4))
                    ttr_v[u, pl.ds(16 * j, 16)] = plsc.load_gather(
                        tin_v, [f_idx, col])
                return c

            lax.fori_loop(0, 32, u_body, 0)
            pltpu.sync_copy(
                ttr_v, out_hbm.at[pl.ds(N_SLABS * (SLAB_W // 2), 32), :])

    return k(weight_t, weight_tail)


def _emb_lookup(idx_flat, table_lin):
    mesh = plsc.VectorSubcoreMesh(core_axis_name="c", subcore_axis_name="s")

    @functools.partial(
        pl.kernel,
        mesh=mesh,
        out_type=jax.ShapeDtypeStruct((TOTAL, EMBED_DIM), jnp.float32),
        scratch_types=[
            pltpu.VMEM((BPW,), jnp.int32),
            pltpu.VMEM((2, CHUNK, EMBED_DIM), jnp.float32),
            pltpu.SemaphoreType.DMA,
            pltpu.SemaphoreType.DMA,
        ],
        compiler_params=pltpu.CompilerParams(use_tc_tiling_on_sc=False),
    )
    def k(idx_hbm, table_hbm, out_hbm, idx_v, rows_v, gsem, wsem):
        wid = lax.axis_index("s") * 2 + lax.axis_index("c")
        base = wid * BPW
        pltpu.sync_copy(idx_hbm.at[pl.ds(base, BPW)], idx_v)

        def gather(c, b):
            return pltpu.async_copy(
                table_hbm.at[idx_v.at[pl.ds(c * CHUNK, CHUNK)]],
                rows_v.at[b], gsem)

        def write(c, b):
            return pltpu.async_copy(
                rows_v.at[b], out_hbm.at[pl.ds(base + c * CHUNK, CHUNK)], wsem)

        writes = [None] * NCHUNK
        g = gather(0, 0)
        for c in range(NCHUNK):
            b = c % 2
            if c + 1 < NCHUNK:
                if c >= 1:
                    writes[c - 1].wait()
                g_next = gather(c + 1, (c + 1) % 2)
            g.wait()
            writes[c] = write(c, b)
            if c + 1 < NCHUNK:
                g = g_next
        writes[NCHUNK - 2].wait()
        writes[NCHUNK - 1].wait()

    return k(idx_flat, table_lin)


def kernel(input, weight):
    idx = input.reshape(-1).astype(jnp.int32)
    wtail = jnp.pad(weight[VOCAB - 64:].T, ((0, 0), (0, 64)))
    table_fused = _transpose_table(weight.T, wtail)
    table_lin = table_fused.reshape(VOCAB, EMBED_DIM)
    out = _emb_lookup(idx, table_lin)
    return out.reshape(input.shape + (EMBED_DIM,))
